# Initial kernel scaffold; baseline (speedup 1.0000x reference)
import jax
import jax.numpy as jnp
from jax.experimental import pallas as pl


def kernel(x, edge_index, edge_weight, W1, b1, gamma, beta, W2, b2):
    raise NotImplementedError("write your pallas kernel here")


# speedup vs baseline: 12.5968x; 12.5968x over previous
import functools

import jax
import jax.numpy as jnp
from jax import lax
from jax.experimental import pallas as pl
from jax.experimental.pallas import tpu as pltpu
from jax.experimental.pallas import tpu_sc as plsc

N = 10000
E = 320000
D = 128
C = 400
CA = 80
NS = 16
NC = 2

_MESH = plsc.VectorSubcoreMesh(core_axis_name="c", subcore_axis_name="s")
_SC_PARAMS = pltpu.CompilerParams(needs_layout_passes=False)


def _rsqrt16(v):
    half = v * 0.5
    iv = lax.bitcast_convert_type(v, jnp.int32)
    iv = jnp.full((16,), 0x5F3759DF, jnp.int32) - (iv >> 1)
    y = lax.bitcast_convert_type(iv, jnp.float32)
    y = y * (1.5 - half * y * y)
    y = y * (1.5 - half * y * y)
    y = y * (1.5 - half * y * y)
    return y


@functools.partial(
    pl.kernel,
    mesh=_MESH,
    compiler_params=_SC_PARAMS,
    out_type=[
        jax.ShapeDtypeStruct((E,), jnp.float32),
        jax.ShapeDtypeStruct((N,), jnp.float32),
    ],
    scratch_types=[
        pltpu.VMEM_SHARED((N,), jnp.float32),
        pltpu.VMEM((N,), jnp.float32),
        pltpu.VMEM((1024,), jnp.float32),
        pltpu.VMEM((C,), jnp.int32),
        pltpu.VMEM((C,), jnp.int32),
        pltpu.VMEM((C,), jnp.float32),
        pltpu.VMEM((C,), jnp.float32),
    ],
)
def _norm_kernel(src_hbm, dst_hbm, ew_hbm, norm_hbm, deg_hbm,
                 deg_acc, deg_t, ones_v, src_v, dst_v, ew_v, norm_v):
    c = lax.axis_index("c")
    s = lax.axis_index("s")

    ones16 = jnp.full((16,), 1.0, jnp.float32)

    def fill_ones(i, carry):
        ones_v[pl.ds(i * 16, 16)] = ones16
        return carry

    lax.fori_loop(0, 64, fill_ones, 0)

    @pl.when(s < 10)
    def _():
        pltpu.sync_copy(ones_v.at[pl.ds(0, 1000)],
                        deg_acc.at[pl.ds(s * 1000, 1000)])

    plsc.subcore_barrier()

    pe = E // NS
    base = s * pe

    def deg_chunk(j, carry):
        off = base + j * C
        pltpu.sync_copy(dst_hbm.at[pl.ds(off, C)], dst_v)
        pltpu.sync_copy(ew_hbm.at[pl.ds(off, C)], ew_v)
        pltpu.sync_copy(ew_v, deg_acc.at[dst_v], add=True)
        return carry

    lax.fori_loop(0, pe // C, deg_chunk, 0)
    plsc.subcore_barrier()

    pltpu.sync_copy(deg_acc, deg_t)

    @pl.when((c == 0) & (s < 10))
    def _():
        pltpu.sync_copy(deg_t.at[pl.ds(s * 1000, 1000)],
                        deg_hbm.at[pl.ds(s * 1000, 1000)])

    def rsq(i, carry):
        deg_t[pl.ds(i * 16, 16)] = _rsqrt16(deg_t[pl.ds(i * 16, 16)])
        return carry

    lax.fori_loop(0, N // 16, rsq, 0)

    w = s * NC + c
    pe2 = E // (NS * NC)
    base2 = w * pe2

    def norm_chunk(j, carry):
        off = base2 + j * C
        pltpu.sync_copy(src_hbm.at[pl.ds(off, C)], src_v)
        pltpu.sync_copy(dst_hbm.at[pl.ds(off, C)], dst_v)
        pltpu.sync_copy(ew_hbm.at[pl.ds(off, C)], ew_v)

        def inner(i, carry2):
            s16 = src_v[pl.ds(i * 16, 16)]
            d16 = dst_v[pl.ds(i * 16, 16)]
            w16 = ew_v[pl.ds(i * 16, 16)]
            g1 = plsc.load_gather(deg_t, [s16])
            g2 = plsc.load_gather(deg_t, [d16])
            norm_v[pl.ds(i * 16, 16)] = g1 * w16 * g2
            return carry2

        lax.fori_loop(0, C // 16, inner, 0)
        pltpu.sync_copy(norm_v, norm_hbm.at[pl.ds(off, C)])
        return carry

    lax.fori_loop(0, pe2 // C, norm_chunk, 0)


@functools.partial(
    pl.kernel,
    mesh=_MESH,
    compiler_params=_SC_PARAMS,
    out_type=jax.ShapeDtypeStruct((NC * N, D), jnp.float32),
    scratch_types=[
        pltpu.VMEM_SHARED((N, D), jnp.float32),
        pltpu.VMEM((CA, D), jnp.float32),
        pltpu.VMEM((CA,), jnp.int32),
        pltpu.VMEM((CA,), jnp.int32),
        pltpu.VMEM((CA,), jnp.float32),
        pltpu.SemaphoreType.DMA,
    ],
)
def _agg_kernel(h_hbm, src_hbm, dst_hbm, norm_hbm, out_hbm,
                acc, rows_v, src_v, dst_v, norm_v, gsem):
    c = lax.axis_index("c")
    s = lax.axis_index("s")

    z16 = jnp.zeros((16,), jnp.float32)

    def zrow(i, carry):
        for j in range(D // 16):
            rows_v[i, pl.ds(j * 16, 16)] = z16
        return carry

    lax.fori_loop(0, CA, zrow, 0)
    rbase = s * 632

    def zacc(j, carry):
        pltpu.sync_copy(rows_v, acc.at[pl.ds(rbase + j * CA, CA)])
        return carry

    lax.fori_loop(0, 6, zacc, 0)

    @pl.when(s < 15)
    def _():
        pltpu.sync_copy(rows_v, acc.at[pl.ds(rbase + 480, 80)])
        pltpu.sync_copy(rows_v.at[pl.ds(0, 72)], acc.at[pl.ds(rbase + 560, 72)])

    @pl.when(s == 15)
    def _():
        pltpu.sync_copy(rows_v.at[pl.ds(0, 40)], acc.at[pl.ds(rbase + 480, 40)])

    plsc.subcore_barrier()

    pe = E // (NS * NC)
    base = (c * NS + s) * pe

    def chunk(g, carry):
        off = base + g * CA
        pltpu.sync_copy(src_hbm.at[pl.ds(off, CA)], src_v)
        cp = pltpu.async_copy(h_hbm.at[src_v], rows_v, gsem)
        pltpu.sync_copy(norm_hbm.at[pl.ds(off, CA)], norm_v)
        pltpu.sync_copy(dst_hbm.at[pl.ds(off, CA)], dst_v)
        cp.wait()

        def scale(i, carry2):
            n16 = norm_v[pl.ds(i * 16, 16)]
            for k in range(16):
                nv = jnp.full((16,), n16[k], jnp.float32)
                r = i * 16 + k
                for j in range(D // 16):
                    rows_v[r, pl.ds(j * 16, 16)] = (
                        rows_v[r, pl.ds(j * 16, 16)] * nv)
            return carry2

        lax.fori_loop(0, CA // 16, scale, 0)
        pltpu.sync_copy(rows_v, acc.at[dst_v], add=True)
        return carry

    lax.fori_loop(0, pe // CA, chunk, 0)
    plsc.subcore_barrier()

    obase = c * N + rbase

    @pl.when(s < 15)
    def _():
        pltpu.sync_copy(acc.at[pl.ds(rbase, 400)], out_hbm.at[pl.ds(obase, 400)])
        pltpu.sync_copy(acc.at[pl.ds(rbase + 400, 232)],
                        out_hbm.at[pl.ds(obase + 400, 232)])

    @pl.when(s == 15)
    def _():
        pltpu.sync_copy(acc.at[pl.ds(rbase, 400)], out_hbm.at[pl.ds(obase, 400)])
        pltpu.sync_copy(acc.at[pl.ds(rbase + 400, 120)],
                        out_hbm.at[pl.ds(obase + 400, 120)])


BROWS = 1000


def _mm_body(x_ref, w_ref, out_ref):
    out_ref[...] = jnp.dot(x_ref[...], w_ref[...],
                           preferred_element_type=jnp.float32)


def _matmul(x, w):
    return pl.pallas_call(
        _mm_body,
        grid=(N // BROWS,),
        in_specs=[
            pl.BlockSpec((BROWS, D), lambda i: (i, 0)),
            pl.BlockSpec((D, D), lambda i: (0, 0)),
        ],
        out_specs=pl.BlockSpec((BROWS, D), lambda i: (i, 0)),
        out_shape=jax.ShapeDtypeStruct((N, D), jnp.float32),
    )(x, w)


def _k4a_body(agg_ref, h_ref, deg_ref, b_ref, z_ref, st_ref):
    a = agg_ref[0] + agg_ref[1]
    z = a + h_ref[...] / deg_ref[...] + b_ref[...]
    z_ref[...] = z

    @pl.when(pl.program_id(0) == 0)
    def _():
        st_ref[...] = jnp.zeros_like(st_ref)

    st_ref[0:1, :] += jnp.sum(z, axis=0, keepdims=True)
    st_ref[1:2, :] += jnp.sum(z * z, axis=0, keepdims=True)


def _k4a(agg, h, deg, b):
    return pl.pallas_call(
        _k4a_body,
        grid=(N // BROWS,),
        in_specs=[
            pl.BlockSpec((2, BROWS, D), lambda i: (0, i, 0)),
            pl.BlockSpec((BROWS, D), lambda i: (i, 0)),
            pl.BlockSpec((BROWS, 1), lambda i: (i, 0)),
            pl.BlockSpec((1, D), lambda i: (0, 0)),
        ],
        out_specs=[
            pl.BlockSpec((BROWS, D), lambda i: (i, 0)),
            pl.BlockSpec((2, D), lambda i: (0, 0)),
        ],
        out_shape=[
            jax.ShapeDtypeStruct((N, D), jnp.float32),
            jax.ShapeDtypeStruct((2, D), jnp.float32),
        ],
    )(agg, h, deg, b)


def _k4b_body(z_ref, st_ref, gam_ref, bet_ref, w_ref, out_ref):
    inv_n = 1.0 / N
    mean = st_ref[0:1, :] * inv_n
    var = st_ref[1:2, :] * inv_n - mean * mean
    xn = (z_ref[...] - mean) * lax.rsqrt(var + 1e-5) * gam_ref[...] + bet_ref[...]
    xn = jnp.maximum(xn, 0.0)
    out_ref[...] = jnp.dot(xn, w_ref[...], preferred_element_type=jnp.float32)


def _k4b(z, st, gam, bet, w):
    return pl.pallas_call(
        _k4b_body,
        grid=(N // BROWS,),
        in_specs=[
            pl.BlockSpec((BROWS, D), lambda i: (i, 0)),
            pl.BlockSpec((2, D), lambda i: (0, 0)),
            pl.BlockSpec((1, D), lambda i: (0, 0)),
            pl.BlockSpec((1, D), lambda i: (0, 0)),
            pl.BlockSpec((D, D), lambda i: (0, 0)),
        ],
        out_specs=pl.BlockSpec((BROWS, D), lambda i: (i, 0)),
        out_shape=jax.ShapeDtypeStruct((N, D), jnp.float32),
    )(z, st, gam, bet, w)


def _k6_body(agg_ref, g_ref, deg_ref, b_ref, out_ref):
    o = agg_ref[0] + agg_ref[1] + g_ref[...] / deg_ref[...] + b_ref[...]
    m = jnp.max(o, axis=1, keepdims=True)
    t = o - m
    lse = jnp.log(jnp.sum(jnp.exp(t), axis=1, keepdims=True))
    out_ref[...] = t - lse


def _k6(agg, g, deg, b):
    return pl.pallas_call(
        _k6_body,
        grid=(N // BROWS,),
        in_specs=[
            pl.BlockSpec((2, BROWS, D), lambda i: (0, i, 0)),
            pl.BlockSpec((BROWS, D), lambda i: (i, 0)),
            pl.BlockSpec((BROWS, 1), lambda i: (i, 0)),
            pl.BlockSpec((1, D), lambda i: (0, 0)),
        ],
        out_specs=pl.BlockSpec((BROWS, D), lambda i: (i, 0)),
        out_shape=jax.ShapeDtypeStruct((N, D), jnp.float32),
    )(agg, g, deg, b)


def kernel(x, edge_index, edge_weight, W1, b1, gamma, beta, W2, b2):
    src = edge_index[0]
    dst = edge_index[1]
    norm, deg = _norm_kernel(src, dst, edge_weight)
    deg2 = deg.reshape(N, 1)

    h1 = _matmul(x, W1)
    agg1 = _agg_kernel(h1, src, dst, norm).reshape(NC, N, D)
    z, st = _k4a(agg1, h1, deg2, b1.reshape(1, D))
    g2 = _k4b(z, st, gamma.reshape(1, D), beta.reshape(1, D), W2)
    agg2 = _agg_kernel(g2, src, dst, norm).reshape(NC, N, D)
    return _k6(agg2, g2, deg2, b2.reshape(1, D))


# speedup vs baseline: 18.9292x; 1.5027x over previous
import functools

import jax
import jax.numpy as jnp
from jax import lax
from jax.experimental import pallas as pl
from jax.experimental.pallas import tpu as pltpu
from jax.experimental.pallas import tpu_sc as plsc

N = 10000
E = 320000
D = 128
C = 400
CA = 80
NS = 16
NC = 2

_MESH = plsc.VectorSubcoreMesh(core_axis_name="c", subcore_axis_name="s")
_SC_PARAMS = pltpu.CompilerParams(needs_layout_passes=False)


def _rsqrt16(v):
    half = v * 0.5
    iv = lax.bitcast_convert_type(v, jnp.int32)
    iv = jnp.full((16,), 0x5F3759DF, jnp.int32) - (iv >> 1)
    y = lax.bitcast_convert_type(iv, jnp.float32)
    y = y * (1.5 - half * y * y)
    y = y * (1.5 - half * y * y)
    y = y * (1.5 - half * y * y)
    return y


@functools.partial(
    pl.kernel,
    mesh=_MESH,
    compiler_params=_SC_PARAMS,
    out_type=[
        jax.ShapeDtypeStruct((E,), jnp.float32),
        jax.ShapeDtypeStruct((N,), jnp.float32),
    ],
    scratch_types=[
        pltpu.VMEM_SHARED((N,), jnp.float32),
        pltpu.VMEM((N,), jnp.float32),
        pltpu.VMEM((1024,), jnp.float32),
        pltpu.VMEM((C,), jnp.int32),
        pltpu.VMEM((C,), jnp.int32),
        pltpu.VMEM((C,), jnp.float32),
        pltpu.VMEM((C,), jnp.float32),
    ],
)
def _norm_kernel(src_hbm, dst_hbm, ew_hbm, norm_hbm, deg_hbm,
                 deg_acc, deg_t, ones_v, src_v, dst_v, ew_v, norm_v):
    c = lax.axis_index("c")
    s = lax.axis_index("s")

    ones16 = jnp.full((16,), 1.0, jnp.float32)

    def fill_ones(i, carry):
        ones_v[pl.ds(i * 16, 16)] = ones16
        return carry

    lax.fori_loop(0, 64, fill_ones, 0)

    @pl.when(s < 10)
    def _():
        pltpu.sync_copy(ones_v.at[pl.ds(0, 1000)],
                        deg_acc.at[pl.ds(s * 1000, 1000)])

    plsc.subcore_barrier()

    pe = E // NS
    base = s * pe

    def deg_chunk(j, carry):
        off = base + j * C
        pltpu.sync_copy(dst_hbm.at[pl.ds(off, C)], dst_v)
        pltpu.sync_copy(ew_hbm.at[pl.ds(off, C)], ew_v)
        pltpu.sync_copy(ew_v, deg_acc.at[dst_v], add=True)
        return carry

    lax.fori_loop(0, pe // C, deg_chunk, 0)
    plsc.subcore_barrier()

    pltpu.sync_copy(deg_acc, deg_t)

    @pl.when((c == 0) & (s < 10))
    def _():
        pltpu.sync_copy(deg_t.at[pl.ds(s * 1000, 1000)],
                        deg_hbm.at[pl.ds(s * 1000, 1000)])

    def rsq(i, carry):
        deg_t[pl.ds(i * 16, 16)] = _rsqrt16(deg_t[pl.ds(i * 16, 16)])
        return carry

    lax.fori_loop(0, N // 16, rsq, 0)

    w = s * NC + c
    pe2 = E // (NS * NC)
    base2 = w * pe2

    def norm_chunk(j, carry):
        off = base2 + j * C
        pltpu.sync_copy(src_hbm.at[pl.ds(off, C)], src_v)
        pltpu.sync_copy(dst_hbm.at[pl.ds(off, C)], dst_v)
        pltpu.sync_copy(ew_hbm.at[pl.ds(off, C)], ew_v)

        def inner(i, carry2):
            s16 = src_v[pl.ds(i * 16, 16)]
            d16 = dst_v[pl.ds(i * 16, 16)]
            w16 = ew_v[pl.ds(i * 16, 16)]
            g1 = plsc.load_gather(deg_t, [s16])
            g2 = plsc.load_gather(deg_t, [d16])
            norm_v[pl.ds(i * 16, 16)] = g1 * w16 * g2
            return carry2

        lax.fori_loop(0, C // 16, inner, 0)
        pltpu.sync_copy(norm_v, norm_hbm.at[pl.ds(off, C)])
        return carry

    lax.fori_loop(0, pe2 // C, norm_chunk, 0)


BLK = 2000


@functools.partial(
    pl.kernel,
    mesh=_MESH,
    compiler_params=_SC_PARAMS,
    out_type=jax.ShapeDtypeStruct((NC * N, D), jnp.float32),
    scratch_types=[
        pltpu.VMEM_SHARED((N, D), jnp.float32),
        [pltpu.VMEM((CA, D), jnp.float32)] * 3,
        pltpu.VMEM((BLK,), jnp.int32),
        pltpu.VMEM((BLK,), jnp.int32),
        pltpu.VMEM((BLK,), jnp.float32),
        [pltpu.VMEM((CA,), jnp.int32)] * 3,
        [pltpu.VMEM((CA,), jnp.int32)] * 3,
        [pltpu.VMEM((CA,), jnp.float32)] * 3,
        [pltpu.SemaphoreType.DMA] * 3,
        [pltpu.SemaphoreType.DMA] * 3,
    ],
)
def _agg_kernel(h_hbm, src_hbm, dst_hbm, norm_hbm, out_hbm,
                acc, rows, src_b, dst_b, norm_b, srcx, dstx, nx, gsem, ssem):
    c = lax.axis_index("c")
    s = lax.axis_index("s")

    z16 = jnp.zeros((16,), jnp.float32)

    def zrow(i, carry):
        for j in range(D // 16):
            rows[0][i, pl.ds(j * 16, 16)] = z16
        return carry

    lax.fori_loop(0, CA, zrow, 0)
    rbase = s * 632

    def zacc(j, carry):
        pltpu.sync_copy(rows[0], acc.at[pl.ds(rbase + j * CA, CA)])
        return carry

    lax.fori_loop(0, 6, zacc, 0)

    @pl.when(s < 15)
    def _():
        pltpu.sync_copy(rows[0], acc.at[pl.ds(rbase + 480, 80)])
        pltpu.sync_copy(rows[0].at[pl.ds(0, 72)], acc.at[pl.ds(rbase + 560, 72)])

    @pl.when(s == 15)
    def _():
        pltpu.sync_copy(rows[0].at[pl.ds(0, 40)], acc.at[pl.ds(rbase + 480, 40)])

    plsc.subcore_barrier()

    pe = E // (NS * NC)
    base = (c * NS + s) * pe
    nch = pe // CA
    cpb = BLK // CA

    def load_block(b):
        off = base + b * BLK
        pltpu.sync_copy(src_hbm.at[pl.ds(off, BLK)], src_b)
        pltpu.sync_copy(dst_hbm.at[pl.ds(off, BLK)], dst_b)
        pltpu.sync_copy(norm_hbm.at[pl.ds(off, BLK)], norm_b)

    def build_src(q, woff):
        for i in range(CA // 16):
            srcx[q][pl.ds(i * 16, 16)] = src_b[pl.ds(woff + i * 16, 16)]

    def build_dst_norm(p, woff):
        for i in range(CA // 16):
            dstx[p][pl.ds(i * 16, 16)] = dst_b[pl.ds(woff + i * 16, 16)]
            nx[p][pl.ds(i * 16, 16)] = norm_b[pl.ds(woff + i * 16, 16)]

    def drain(sem, slot):
        pltpu.make_async_copy(h_hbm.at[pl.ds(0, CA)], rows[slot], sem).wait()

    def scale(p):
        def body(i, carry):
            n16 = nx[p][pl.ds(i * 16, 16)]
            for k in range(16):
                nv = jnp.full((16,), n16[k], jnp.float32)
                r = i * 16 + k
                for j in range(D // 16):
                    rows[p][r, pl.ds(j * 16, 16)] = (
                        rows[p][r, pl.ds(j * 16, 16)] * nv)
            return carry

        lax.fori_loop(0, CA // 16, body, 0)

    def chunk_step(g, p, q, wait_scatter):
        woff = lax.rem(g, cpb) * CA
        drain(gsem[p], p)
        build_dst_norm(p, woff)
        if wait_scatter:
            drain(ssem[q], q)
        gn = g + 1

        @pl.when(gn < nch)
        def _():
            @pl.when(lax.rem(gn, cpb) == 0)
            def _():
                load_block(gn // cpb)

            build_src(q, lax.rem(gn, cpb) * CA)
            pltpu.async_copy(h_hbm.at[srcx[q]], rows[q], gsem[q])

        scale(p)
        pltpu.async_copy(rows[p], acc.at[dstx[p]], ssem[p], add=True)

    load_block(0)
    build_src(0, 0)
    pltpu.async_copy(h_hbm.at[srcx[0]], rows[0], gsem[0])
    chunk_step(0, 0, 1, wait_scatter=False)
    chunk_step(1, 1, 2, wait_scatter=False)

    def triple(t, carry):
        g = 3 * t + 2
        chunk_step(g, 2, 0, wait_scatter=True)
        chunk_step(g + 1, 0, 1, wait_scatter=True)
        chunk_step(g + 2, 1, 2, wait_scatter=True)
        return carry

    lax.fori_loop(0, (nch - 2) // 3, triple, 0)
    drain(ssem[0], 0)
    drain(ssem[1], 1)
    plsc.subcore_barrier()

    obase = c * N + rbase

    @pl.when(s < 15)
    def _():
        pltpu.sync_copy(acc.at[pl.ds(rbase, 400)], out_hbm.at[pl.ds(obase, 400)])
        pltpu.sync_copy(acc.at[pl.ds(rbase + 400, 232)],
                        out_hbm.at[pl.ds(obase + 400, 232)])

    @pl.when(s == 15)
    def _():
        pltpu.sync_copy(acc.at[pl.ds(rbase, 400)], out_hbm.at[pl.ds(obase, 400)])
        pltpu.sync_copy(acc.at[pl.ds(rbase + 400, 120)],
                        out_hbm.at[pl.ds(obase + 400, 120)])


BROWS = 1000


def _mm_body(x_ref, w_ref, out_ref):
    out_ref[...] = jnp.dot(x_ref[...], w_ref[...],
                           preferred_element_type=jnp.float32)


def _matmul(x, w):
    return pl.pallas_call(
        _mm_body,
        grid=(N // BROWS,),
        in_specs=[
            pl.BlockSpec((BROWS, D), lambda i: (i, 0)),
            pl.BlockSpec((D, D), lambda i: (0, 0)),
        ],
        out_specs=pl.BlockSpec((BROWS, D), lambda i: (i, 0)),
        out_shape=jax.ShapeDtypeStruct((N, D), jnp.float32),
    )(x, w)


def _k4a_body(agg_ref, h_ref, deg_ref, b_ref, z_ref, st_ref):
    a = agg_ref[0] + agg_ref[1]
    z = a + h_ref[...] / deg_ref[...] + b_ref[...]
    z_ref[...] = z

    @pl.when(pl.program_id(0) == 0)
    def _():
        st_ref[...] = jnp.zeros_like(st_ref)

    st_ref[0:1, :] += jnp.sum(z, axis=0, keepdims=True)
    st_ref[1:2, :] += jnp.sum(z * z, axis=0, keepdims=True)


def _k4a(agg, h, deg, b):
    return pl.pallas_call(
        _k4a_body,
        grid=(N // BROWS,),
        in_specs=[
            pl.BlockSpec((2, BROWS, D), lambda i: (0, i, 0)),
            pl.BlockSpec((BROWS, D), lambda i: (i, 0)),
            pl.BlockSpec((BROWS, 1), lambda i: (i, 0)),
            pl.BlockSpec((1, D), lambda i: (0, 0)),
        ],
        out_specs=[
            pl.BlockSpec((BROWS, D), lambda i: (i, 0)),
            pl.BlockSpec((2, D), lambda i: (0, 0)),
        ],
        out_shape=[
            jax.ShapeDtypeStruct((N, D), jnp.float32),
            jax.ShapeDtypeStruct((2, D), jnp.float32),
        ],
    )(agg, h, deg, b)


def _k4b_body(z_ref, st_ref, gam_ref, bet_ref, w_ref, out_ref):
    inv_n = 1.0 / N
    mean = st_ref[0:1, :] * inv_n
    var = st_ref[1:2, :] * inv_n - mean * mean
    xn = (z_ref[...] - mean) * lax.rsqrt(var + 1e-5) * gam_ref[...] + bet_ref[...]
    xn = jnp.maximum(xn, 0.0)
    out_ref[...] = jnp.dot(xn, w_ref[...], preferred_element_type=jnp.float32)


def _k4b(z, st, gam, bet, w):
    return pl.pallas_call(
        _k4b_body,
        grid=(N // BROWS,),
        in_specs=[
            pl.BlockSpec((BROWS, D), lambda i: (i, 0)),
            pl.BlockSpec((2, D), lambda i: (0, 0)),
            pl.BlockSpec((1, D), lambda i: (0, 0)),
            pl.BlockSpec((1, D), lambda i: (0, 0)),
            pl.BlockSpec((D, D), lambda i: (0, 0)),
        ],
        out_specs=pl.BlockSpec((BROWS, D), lambda i: (i, 0)),
        out_shape=jax.ShapeDtypeStruct((N, D), jnp.float32),
    )(z, st, gam, bet, w)


def _k6_body(agg_ref, g_ref, deg_ref, b_ref, out_ref):
    o = agg_ref[0] + agg_ref[1] + g_ref[...] / deg_ref[...] + b_ref[...]
    m = jnp.max(o, axis=1, keepdims=True)
    t = o - m
    lse = jnp.log(jnp.sum(jnp.exp(t), axis=1, keepdims=True))
    out_ref[...] = t - lse


def _k6(agg, g, deg, b):
    return pl.pallas_call(
        _k6_body,
        grid=(N // BROWS,),
        in_specs=[
            pl.BlockSpec((2, BROWS, D), lambda i: (0, i, 0)),
            pl.BlockSpec((BROWS, D), lambda i: (i, 0)),
            pl.BlockSpec((BROWS, 1), lambda i: (i, 0)),
            pl.BlockSpec((1, D), lambda i: (0, 0)),
        ],
        out_specs=pl.BlockSpec((BROWS, D), lambda i: (i, 0)),
        out_shape=jax.ShapeDtypeStruct((N, D), jnp.float32),
    )(agg, g, deg, b)


def kernel(x, edge_index, edge_weight, W1, b1, gamma, beta, W2, b2):
    src = edge_index[0]
    dst = edge_index[1]
    norm, deg = _norm_kernel(src, dst, edge_weight)
    deg2 = deg.reshape(N, 1)

    h1 = _matmul(x, W1)
    agg1 = _agg_kernel(h1, src, dst, norm).reshape(NC, N, D)
    z, st = _k4a(agg1, h1, deg2, b1.reshape(1, D))
    g2 = _k4b(z, st, gamma.reshape(1, D), beta.reshape(1, D), W2)
    agg2 = _agg_kernel(g2, src, dst, norm).reshape(NC, N, D)
    return _k6(agg2, g2, deg2, b2.reshape(1, D))


# speedup vs baseline: 22.8824x; 1.2088x over previous
import functools

import jax
import jax.numpy as jnp
from jax import lax
from jax.experimental import pallas as pl
from jax.experimental.pallas import tpu as pltpu
from jax.experimental.pallas import tpu_sc as plsc

N = 10000
E = 320000
D = 128
C = 400
CA = 80
NS = 16
NC = 2

_MESH = plsc.VectorSubcoreMesh(core_axis_name="c", subcore_axis_name="s")
_SC_PARAMS = pltpu.CompilerParams(needs_layout_passes=False)


def _rsqrt16(v):
    half = v * 0.5
    iv = lax.bitcast_convert_type(v, jnp.int32)
    iv = jnp.full((16,), 0x5F3759DF, jnp.int32) - (iv >> 1)
    y = lax.bitcast_convert_type(iv, jnp.float32)
    y = y * (1.5 - half * y * y)
    y = y * (1.5 - half * y * y)
    y = y * (1.5 - half * y * y)
    return y


PEA = E // NS
PEC = E // (NS * NC)


@functools.partial(
    pl.kernel,
    mesh=_MESH,
    compiler_params=_SC_PARAMS,
    out_type=[
        jax.ShapeDtypeStruct((E,), jnp.float32),
        jax.ShapeDtypeStruct((N,), jnp.float32),
    ],
    scratch_types=[
        pltpu.VMEM_SHARED((N,), jnp.float32),
        pltpu.VMEM((N,), jnp.float32),
        pltpu.VMEM((PEA,), jnp.int32),
        pltpu.VMEM((PEA,), jnp.float32),
        pltpu.VMEM((PEC,), jnp.int32),
        pltpu.VMEM((PEC,), jnp.float32),
        pltpu.VMEM((1024,), jnp.float32),
        [pltpu.VMEM((C,), jnp.int32)] * 3,
        [pltpu.SemaphoreType.DMA] * 3,
        pltpu.SemaphoreType.DMA,
    ],
)
def _norm_kernel(src_hbm, dst_hbm, ew_hbm, norm_hbm, deg_hbm,
                 deg_acc, deg_t, dst_s, ew_s, src_s, norm_s, ones_v,
                 dstx, ssem, lsem):
    c = lax.axis_index("c")
    s = lax.axis_index("s")

    base_a = s * PEA
    base_c = base_a + c * PEC
    cp1 = pltpu.async_copy(dst_hbm.at[pl.ds(base_a, PEA)], dst_s, lsem)
    cp2 = pltpu.async_copy(ew_hbm.at[pl.ds(base_a, PEA)], ew_s, lsem)
    cp3 = pltpu.async_copy(src_hbm.at[pl.ds(base_c, PEC)], src_s, lsem)

    ones16 = jnp.full((16,), 1.0, jnp.float32)

    def fill_ones(i, carry):
        ones_v[pl.ds(i * 16, 16)] = ones16
        return carry

    lax.fori_loop(0, 64, fill_ones, 0)

    @pl.when(s < 10)
    def _():
        pltpu.sync_copy(ones_v.at[pl.ds(0, 1000)],
                        deg_acc.at[pl.ds(s * 1000, 1000)])

    cp1.wait()
    cp2.wait()
    cp3.wait()
    plsc.subcore_barrier()

    def build_dstx(slot, off):
        for i in range(C // 16):
            dstx[slot][pl.ds(i * 16, 16)] = dst_s[pl.ds(off + i * 16, 16)]

    def drain_s(slot):
        pltpu.make_async_copy(dst_hbm.at[pl.ds(0, C)], dstx[slot],
                              ssem[slot]).wait()

    def step_a(j, slot, wait_prev):
        if wait_prev:
            drain_s(slot)
        build_dstx(slot, j * C)
        pltpu.async_copy(ew_s.at[pl.ds(j * C, C)], deg_acc.at[dstx[slot]],
                         ssem[slot], add=True)

    ncha = PEA // C
    step_a(0, 0, False)
    step_a(1, 1, False)
    step_a(2, 2, False)

    def triple_a(t, carry):
        j = 3 * t + 3
        step_a(j, 0, True)
        step_a(j + 1, 1, True)
        step_a(j + 2, 2, True)
        return carry

    lax.fori_loop(0, (ncha - 3) // 3, triple_a, 0)
    step_a(ncha - 2, 0, True)
    step_a(ncha - 1, 1, True)
    drain_s(0)
    drain_s(1)
    drain_s(2)
    plsc.subcore_barrier()

    pltpu.sync_copy(deg_acc, deg_t)

    @pl.when((c == 0) & (s < 10))
    def _():
        pltpu.sync_copy(deg_t.at[pl.ds(s * 1000, 1000)],
                        deg_hbm.at[pl.ds(s * 1000, 1000)])

    def rsq(i, carry):
        deg_t[pl.ds(i * 16, 16)] = _rsqrt16(deg_t[pl.ds(i * 16, 16)])
        return carry

    lax.fori_loop(0, N // 16, rsq, 0)

    coff = c * PEC

    def normc(i, carry):
        s16 = src_s[pl.ds(i * 16, 16)]
        d16 = dst_s[pl.ds(coff + i * 16, 16)]
        w16 = ew_s[pl.ds(coff + i * 16, 16)]
        g1 = plsc.load_gather(deg_t, [s16])
        g2 = plsc.load_gather(deg_t, [d16])
        norm_s[pl.ds(i * 16, 16)] = g1 * w16 * g2
        return carry

    lax.fori_loop(0, PEC // 16, normc, 0)
    pltpu.sync_copy(norm_s, norm_hbm.at[pl.ds(base_c, PEC)])


BLK = 2000


@functools.partial(
    pl.kernel,
    mesh=_MESH,
    compiler_params=_SC_PARAMS,
    out_type=jax.ShapeDtypeStruct((NC * N, D), jnp.float32),
    scratch_types=[
        pltpu.VMEM_SHARED((N, D), jnp.float32),
        [pltpu.VMEM((CA, D), jnp.float32)] * 3,
        pltpu.VMEM((BLK,), jnp.int32),
        pltpu.VMEM((BLK,), jnp.int32),
        pltpu.VMEM((BLK,), jnp.float32),
        [pltpu.VMEM((CA,), jnp.int32)] * 3,
        [pltpu.VMEM((CA,), jnp.int32)] * 3,
        [pltpu.VMEM((CA,), jnp.float32)] * 3,
        [pltpu.SemaphoreType.DMA] * 3,
        [pltpu.SemaphoreType.DMA] * 3,
    ],
)
def _agg_kernel(h_hbm, src_hbm, dst_hbm, norm_hbm, out_hbm,
                acc, rows, src_b, dst_b, norm_b, srcx, dstx, nx, gsem, ssem):
    c = lax.axis_index("c")
    s = lax.axis_index("s")

    z16 = jnp.zeros((16,), jnp.float32)

    def zrow(i, carry):
        for j in range(D // 16):
            rows[0][i, pl.ds(j * 16, 16)] = z16
        return carry

    lax.fori_loop(0, CA, zrow, 0)
    rbase = s * 632

    def zacc(j, carry):
        pltpu.sync_copy(rows[0], acc.at[pl.ds(rbase + j * CA, CA)])
        return carry

    lax.fori_loop(0, 6, zacc, 0)

    @pl.when(s < 15)
    def _():
        pltpu.sync_copy(rows[0], acc.at[pl.ds(rbase + 480, 80)])
        pltpu.sync_copy(rows[0].at[pl.ds(0, 72)], acc.at[pl.ds(rbase + 560, 72)])

    @pl.when(s == 15)
    def _():
        pltpu.sync_copy(rows[0].at[pl.ds(0, 40)], acc.at[pl.ds(rbase + 480, 40)])

    plsc.subcore_barrier()

    pe = E // (NS * NC)
    base = (c * NS + s) * pe
    nch = pe // CA
    cpb = BLK // CA

    def load_block(b):
        off = base + b * BLK
        pltpu.sync_copy(src_hbm.at[pl.ds(off, BLK)], src_b)
        pltpu.sync_copy(dst_hbm.at[pl.ds(off, BLK)], dst_b)
        pltpu.sync_copy(norm_hbm.at[pl.ds(off, BLK)], norm_b)

    def build_src(q, woff):
        for i in range(CA // 16):
            srcx[q][pl.ds(i * 16, 16)] = src_b[pl.ds(woff + i * 16, 16)]

    def build_dst_norm(p, woff):
        for i in range(CA // 16):
            dstx[p][pl.ds(i * 16, 16)] = dst_b[pl.ds(woff + i * 16, 16)]
            nx[p][pl.ds(i * 16, 16)] = norm_b[pl.ds(woff + i * 16, 16)]

    def drain(sem, slot):
        pltpu.make_async_copy(h_hbm.at[pl.ds(0, CA)], rows[slot], sem).wait()

    def scale(p):
        def body(i, carry):
            n16 = nx[p][pl.ds(i * 16, 16)]
            for k in range(16):
                nv = jnp.full((16,), n16[k], jnp.float32)
                r = i * 16 + k
                for j in range(D // 16):
                    rows[p][r, pl.ds(j * 16, 16)] = (
                        rows[p][r, pl.ds(j * 16, 16)] * nv)
            return carry

        lax.fori_loop(0, CA // 16, body, 0)

    def chunk_step(g, p, q, wait_scatter):
        woff = lax.rem(g, cpb) * CA
        drain(gsem[p], p)
        build_dst_norm(p, woff)
        if wait_scatter:
            drain(ssem[q], q)
        gn = g + 1

        @pl.when(gn < nch)
        def _():
            @pl.when(lax.rem(gn, cpb) == 0)
            def _():
                load_block(gn // cpb)

            build_src(q, lax.rem(gn, cpb) * CA)
            pltpu.async_copy(h_hbm.at[srcx[q]], rows[q], gsem[q])

        scale(p)
        pltpu.async_copy(rows[p], acc.at[dstx[p]], ssem[p], add=True)

    load_block(0)
    build_src(0, 0)
    pltpu.async_copy(h_hbm.at[srcx[0]], rows[0], gsem[0])
    chunk_step(0, 0, 1, wait_scatter=False)
    chunk_step(1, 1, 2, wait_scatter=False)

    def triple(t, carry):
        g = 3 * t + 2
        chunk_step(g, 2, 0, wait_scatter=True)
        chunk_step(g + 1, 0, 1, wait_scatter=True)
        chunk_step(g + 2, 1, 2, wait_scatter=True)
        return carry

    lax.fori_loop(0, (nch - 2) // 3, triple, 0)
    drain(ssem[0], 0)
    drain(ssem[1], 1)
    plsc.subcore_barrier()

    obase = c * N + rbase

    @pl.when(s < 15)
    def _():
        pltpu.sync_copy(acc.at[pl.ds(rbase, 400)], out_hbm.at[pl.ds(obase, 400)])
        pltpu.sync_copy(acc.at[pl.ds(rbase + 400, 232)],
                        out_hbm.at[pl.ds(obase + 400, 232)])

    @pl.when(s == 15)
    def _():
        pltpu.sync_copy(acc.at[pl.ds(rbase, 400)], out_hbm.at[pl.ds(obase, 400)])
        pltpu.sync_copy(acc.at[pl.ds(rbase + 400, 120)],
                        out_hbm.at[pl.ds(obase + 400, 120)])


BROWS = 1000


def _mm_body(x_ref, w_ref, out_ref):
    out_ref[...] = jnp.dot(x_ref[...], w_ref[...],
                           preferred_element_type=jnp.float32)


def _matmul(x, w):
    return pl.pallas_call(
        _mm_body,
        grid=(N // BROWS,),
        in_specs=[
            pl.BlockSpec((BROWS, D), lambda i: (i, 0)),
            pl.BlockSpec((D, D), lambda i: (0, 0)),
        ],
        out_specs=pl.BlockSpec((BROWS, D), lambda i: (i, 0)),
        out_shape=jax.ShapeDtypeStruct((N, D), jnp.float32),
    )(x, w)


def _k4a_body(agg_ref, h_ref, deg_ref, b_ref, z_ref, st_ref):
    a = agg_ref[0] + agg_ref[1]
    z = a + h_ref[...] / deg_ref[...] + b_ref[...]
    z_ref[...] = z

    @pl.when(pl.program_id(0) == 0)
    def _():
        st_ref[...] = jnp.zeros_like(st_ref)

    st_ref[0:1, :] += jnp.sum(z, axis=0, keepdims=True)
    st_ref[1:2, :] += jnp.sum(z * z, axis=0, keepdims=True)


def _k4a(agg, h, deg, b):
    return pl.pallas_call(
        _k4a_body,
        grid=(N // BROWS,),
        in_specs=[
            pl.BlockSpec((2, BROWS, D), lambda i: (0, i, 0)),
            pl.BlockSpec((BROWS, D), lambda i: (i, 0)),
            pl.BlockSpec((BROWS, 1), lambda i: (i, 0)),
            pl.BlockSpec((1, D), lambda i: (0, 0)),
        ],
        out_specs=[
            pl.BlockSpec((BROWS, D), lambda i: (i, 0)),
            pl.BlockSpec((2, D), lambda i: (0, 0)),
        ],
        out_shape=[
            jax.ShapeDtypeStruct((N, D), jnp.float32),
            jax.ShapeDtypeStruct((2, D), jnp.float32),
        ],
    )(agg, h, deg, b)


def _k4b_body(z_ref, st_ref, gam_ref, bet_ref, w_ref, out_ref):
    inv_n = 1.0 / N
    mean = st_ref[0:1, :] * inv_n
    var = st_ref[1:2, :] * inv_n - mean * mean
    xn = (z_ref[...] - mean) * lax.rsqrt(var + 1e-5) * gam_ref[...] + bet_ref[...]
    xn = jnp.maximum(xn, 0.0)
    out_ref[...] = jnp.dot(xn, w_ref[...], preferred_element_type=jnp.float32)


def _k4b(z, st, gam, bet, w):
    return pl.pallas_call(
        _k4b_body,
        grid=(N // BROWS,),
        in_specs=[
            pl.BlockSpec((BROWS, D), lambda i: (i, 0)),
            pl.BlockSpec((2, D), lambda i: (0, 0)),
            pl.BlockSpec((1, D), lambda i: (0, 0)),
            pl.BlockSpec((1, D), lambda i: (0, 0)),
            pl.BlockSpec((D, D), lambda i: (0, 0)),
        ],
        out_specs=pl.BlockSpec((BROWS, D), lambda i: (i, 0)),
        out_shape=jax.ShapeDtypeStruct((N, D), jnp.float32),
    )(z, st, gam, bet, w)


def _k6_body(agg_ref, g_ref, deg_ref, b_ref, out_ref):
    o = agg_ref[0] + agg_ref[1] + g_ref[...] / deg_ref[...] + b_ref[...]
    m = jnp.max(o, axis=1, keepdims=True)
    t = o - m
    lse = jnp.log(jnp.sum(jnp.exp(t), axis=1, keepdims=True))
    out_ref[...] = t - lse


def _k6(agg, g, deg, b):
    return pl.pallas_call(
        _k6_body,
        grid=(N // BROWS,),
        in_specs=[
            pl.BlockSpec((2, BROWS, D), lambda i: (0, i, 0)),
            pl.BlockSpec((BROWS, D), lambda i: (i, 0)),
            pl.BlockSpec((BROWS, 1), lambda i: (i, 0)),
            pl.BlockSpec((1, D), lambda i: (0, 0)),
        ],
        out_specs=pl.BlockSpec((BROWS, D), lambda i: (i, 0)),
        out_shape=jax.ShapeDtypeStruct((N, D), jnp.float32),
    )(agg, g, deg, b)


def kernel(x, edge_index, edge_weight, W1, b1, gamma, beta, W2, b2):
    src = edge_index[0]
    dst = edge_index[1]
    norm, deg = _norm_kernel(src, dst, edge_weight)
    deg2 = deg.reshape(N, 1)

    h1 = _matmul(x, W1)
    agg1 = _agg_kernel(h1, src, dst, norm).reshape(NC, N, D)
    z, st = _k4a(agg1, h1, deg2, b1.reshape(1, D))
    g2 = _k4b(z, st, gamma.reshape(1, D), beta.reshape(1, D), W2)
    agg2 = _agg_kernel(g2, src, dst, norm).reshape(NC, N, D)
    return _k6(agg2, g2, deg2, b2.reshape(1, D))
